# baseline (speedup 1.0000x reference)
import functools

import jax
import jax.numpy as jnp
from jax import lax
from jax.experimental import pallas as pl
from jax.experimental.pallas import tpu as pltpu
from jax.experimental.pallas import tpu_sc as plsc

_F32 = jnp.float32


def _tc_body(ids_ref, embt_ref, pw_ref, b_ref, log_ref):
    lblk = ids_ref.shape[0]
    nb = ids_ref.shape[1]

    embt = embt_ref[...]
    ltabt = jnp.dot(pw_ref[...], embt, preferred_element_type=_F32) + b_ref[...]

    iota_v = jax.lax.broadcasted_iota(jnp.int32, (16, nb), 0)
    for l in range(lblk):
        idrow = ids_ref[pl.ds(l, 1), :]
        oh = (jnp.broadcast_to(idrow, (16, nb)) == iota_v).astype(_F32)
        log_ref[l, :, :] = jnp.dot(ltabt, oh, preferred_element_type=_F32)


def _sc_body(b_per_w, seq, ids_hbm, lut_hbm, out_hbm, ids_v, lut_v, out_v, sem):
    nc = 2
    wid = lax.axis_index("s") * nc + lax.axis_index("c")
    b0 = wid * b_per_w
    ngrp = b_per_w // 16

    pltpu.sync_copy(lut_hbm, lut_v)

    def l_body(l, carry):
        pltpu.sync_copy(ids_hbm.at[l, pl.ds(b0, b_per_w)], ids_v)

        def g_body(g, c):
            idvec = ids_v[pl.ds(g * 16, 16)]
            for d in range(16):
                lrow = lut_v[pl.ds(16 * d, 16)]
                out_v[d, pl.ds(g * 16, 16)] = lrow.at[idvec].get(
                    mode="promise_in_bounds")
            return c

        lax.fori_loop(0, ngrp, g_body, 0, unroll=False)
        pltpu.async_copy(out_v, out_hbm.at[l, :, pl.ds(b0, b_per_w)], sem).wait()
        return carry

    lax.fori_loop(0, seq, l_body, 0, unroll=False)


def kernel(input_ids, embed_table, proj_w, proj_b):
    bsz, seq = input_ids.shape
    d = embed_table.shape[1]
    ids_t = input_ids.T.astype(jnp.int32)
    embt = embed_table.T
    b_col = proj_b.reshape(d, 1)

    lblk = 8
    grid = seq // lblk

    log_t, = pl.pallas_call(
        _tc_body,
        grid=(grid,),
        in_specs=[
            pl.BlockSpec((lblk, bsz), lambda i: (i, 0)),
            pl.BlockSpec((d, d), lambda i: (0, 0)),
            pl.BlockSpec((d, d), lambda i: (0, 0)),
            pl.BlockSpec((d, 1), lambda i: (0, 0)),
        ],
        out_specs=[
            pl.BlockSpec((lblk, d, bsz), lambda i: (i, 0, 0)),
        ],
        out_shape=[
            jax.ShapeDtypeStruct((seq, d, bsz), _F32),
        ],
    )(ids_t, embt, proj_w, b_col)

    info = plsc.get_sparse_core_info()
    nw = info.num_cores * info.num_subcores
    b_per_w = bsz // nw
    mesh = plsc.VectorSubcoreMesh(core_axis_name="c", subcore_axis_name="s")
    lut = embt.reshape(d * d)

    sc_fn = pl.kernel(
        functools.partial(_sc_body, b_per_w, seq),
        mesh=mesh,
        out_type=jax.ShapeDtypeStruct((seq, d, bsz), _F32),
        scratch_types=[
            pltpu.VMEM((b_per_w,), jnp.int32),
            pltpu.VMEM((d * d,), _F32),
            pltpu.VMEM((d, b_per_w), _F32),
            pltpu.SemaphoreType.DMA,
        ],
    )
    hid_t = sc_fn(ids_t, lut)

    hidden = jnp.transpose(hid_t, (2, 0, 1))
    logits = jnp.transpose(log_t, (2, 0, 1))
    return (logits, hidden)


# speedup vs baseline: 3.2451x; 3.2451x over previous
import functools

import jax
import jax.numpy as jnp
from jax import lax
from jax.experimental import pallas as pl
from jax.experimental.pallas import tpu as pltpu
from jax.experimental.pallas import tpu_sc as plsc

_F32 = jnp.float32


def _tc_body(ids_ref, embt_ref, pw_ref, b_ref, log_ref):
    lblk = ids_ref.shape[0]
    nb = ids_ref.shape[1]

    embt = embt_ref[...]
    ltabt = jnp.dot(pw_ref[...], embt, preferred_element_type=_F32) + b_ref[...]

    iota_v = jax.lax.broadcasted_iota(jnp.int32, (16, nb), 0)
    for l in range(lblk):
        idrow = ids_ref[pl.ds(l, 1), :]
        oh = (jnp.broadcast_to(idrow, (16, nb)) == iota_v).astype(_F32)
        log_ref[l, :, :] = jnp.dot(ltabt, oh, preferred_element_type=_F32)


def _sc_body(b_per_w, seq, ids_hbm, lut_hbm, out_hbm, ids_v, lut_v,
             out_v0, out_v1, sem0, sem1):
    nc = 2
    wid = lax.axis_index("s") * nc + lax.axis_index("c")
    b0 = wid * b_per_w
    ngrp = b_per_w // 16

    pltpu.sync_copy(lut_hbm, lut_v)
    lrows = [lut_v[pl.ds(16 * d, 16)] for d in range(16)]
    bufs = (out_v0, out_v1)
    sems = (sem0, sem1)

    def compute(b, out_v):
        def g_body(g, c):
            idvec = ids_v[b, pl.ds(g * 16, 16)]
            for d in range(16):
                out_v[d, pl.ds(g * 16, 16)] = lrows[d].at[idvec].get(
                    mode="promise_in_bounds")
            return c

        lax.fori_loop(0, ngrp, g_body, 0, unroll=4)

    def blk_body(i, carry):
        pltpu.sync_copy(ids_hbm.at[pl.ds(2 * i, 2), pl.ds(b0, b_per_w)], ids_v)
        for b in range(2):
            l = 2 * i + b

            @pl.when(i > 0)
            def _wait_prev():
                pltpu.make_async_copy(
                    bufs[b], out_hbm.at[l, :, pl.ds(b0, b_per_w)], sems[b]
                ).wait()

            compute(b, bufs[b])
            pltpu.async_copy(
                bufs[b], out_hbm.at[l, :, pl.ds(b0, b_per_w)], sems[b])
        return carry

    lax.fori_loop(0, seq // 2, blk_body, 0, unroll=False)
    for b in range(2):
        pltpu.make_async_copy(
            bufs[b], out_hbm.at[seq - 2 + b, :, pl.ds(b0, b_per_w)], sems[b]
        ).wait()


def kernel(input_ids, embed_table, proj_w, proj_b):
    bsz, seq = input_ids.shape
    d = embed_table.shape[1]
    ids_t = input_ids.T.astype(jnp.int32)
    embt = embed_table.T
    b_col = proj_b.reshape(d, 1)

    lblk = 8
    grid = seq // lblk

    log_t, = pl.pallas_call(
        _tc_body,
        grid=(grid,),
        in_specs=[
            pl.BlockSpec((lblk, bsz), lambda i: (i, 0)),
            pl.BlockSpec((d, d), lambda i: (0, 0)),
            pl.BlockSpec((d, d), lambda i: (0, 0)),
            pl.BlockSpec((d, 1), lambda i: (0, 0)),
        ],
        out_specs=[
            pl.BlockSpec((lblk, d, bsz), lambda i: (i, 0, 0)),
        ],
        out_shape=[
            jax.ShapeDtypeStruct((seq, d, bsz), _F32),
        ],
    )(ids_t, embt, proj_w, b_col)

    info = plsc.get_sparse_core_info()
    nw = info.num_cores * info.num_subcores
    b_per_w = bsz // nw
    mesh = plsc.VectorSubcoreMesh(core_axis_name="c", subcore_axis_name="s")
    lut = embt.reshape(d * d)

    sc_fn = pl.kernel(
        functools.partial(_sc_body, b_per_w, seq),
        mesh=mesh,
        out_type=jax.ShapeDtypeStruct((seq, d, bsz), _F32),
        scratch_types=[
            pltpu.VMEM((2, b_per_w), jnp.int32),
            pltpu.VMEM((d * d,), _F32),
            pltpu.VMEM((d, b_per_w), _F32),
            pltpu.VMEM((d, b_per_w), _F32),
            pltpu.SemaphoreType.DMA,
            pltpu.SemaphoreType.DMA,
        ],
    )
    hid_t = sc_fn(ids_t, lut)

    hidden = jnp.transpose(hid_t, (2, 0, 1))
    logits = jnp.transpose(log_t, (2, 0, 1))
    return (logits, hidden)


# speedup vs baseline: 4.0564x; 1.2500x over previous
import functools

import jax
import jax.numpy as jnp
from jax import lax
from jax.experimental import pallas as pl
from jax.experimental.pallas import tpu as pltpu
from jax.experimental.pallas import tpu_sc as plsc

_F32 = jnp.float32


def _tc_body(ids_ref, embt_ref, pw_ref, b_ref, log_ref):
    lblk = ids_ref.shape[0]
    nb = ids_ref.shape[1]

    embt = embt_ref[...]
    ltabt = jnp.dot(pw_ref[...], embt, preferred_element_type=_F32) + b_ref[...]

    iota_v = jax.lax.broadcasted_iota(jnp.int32, (16, nb), 0)
    for l in range(lblk):
        idrow = ids_ref[pl.ds(l, 1), :]
        oh = (jnp.broadcast_to(idrow, (16, nb)) == iota_v).astype(_F32)
        log_ref[l, :, :] = jnp.dot(ltabt, oh, preferred_element_type=_F32)


def _sc_body(b_per_w, seq, ids_hbm, lut_hbm, out_hbm, ids_v0, ids_v1, lut_v,
             out_v0, out_v1, out_v2, out_v3, isem0, isem1,
             sem0, sem1, sem2, sem3):
    nc = 2
    wid = lax.axis_index("s") * nc + lax.axis_index("c")
    b0 = wid * b_per_w
    ngrp = b_per_w // 16
    nsup = seq // 8

    pltpu.sync_copy(lut_hbm, lut_v)
    lrows = [lut_v[pl.ds(16 * d, 16)] for d in range(16)]
    obufs = (out_v0, out_v1, out_v2, out_v3)
    osems = (sem0, sem1, sem2, sem3)
    ibufs = (ids_v0, ids_v1)
    isems = (isem0, isem1)

    def ids_copy(blk4, j):
        return pltpu.make_async_copy(
            ids_hbm.at[pl.ds(blk4 * 4, 4), pl.ds(b0, b_per_w)],
            ibufs[j], isems[j])

    def out_copy(l, b):
        return pltpu.make_async_copy(
            obufs[b], out_hbm.at[l, :, pl.ds(b0, b_per_w)], osems[b])

    def compute(ids_v, b, out_v):
        def g_body(g, c):
            idvec = ids_v[b, pl.ds(g * 16, 16)]
            for d in range(16):
                out_v[d, pl.ds(g * 16, 16)] = lrows[d].at[idvec].get(
                    mode="promise_in_bounds")
            return c

        lax.fori_loop(0, ngrp, g_body, 0, unroll=4)

    ids_copy(0, 0).start()

    def sup_body(i, carry):
        for j in range(2):
            blk4 = 2 * i + j
            ids_copy(blk4, j).wait()

            @pl.when(blk4 + 1 < 2 * nsup)
            def _prefetch():
                ids_copy(blk4 + 1, 1 - j).start()

            for b4 in range(4):
                buf = b4
                l = blk4 * 4 + b4

                @pl.when(blk4 > 0)
                def _wait_prev():
                    out_copy(l, buf).wait()

                compute(ibufs[j], b4, obufs[buf])
                out_copy(l, buf).start()
        return carry

    lax.fori_loop(0, nsup, sup_body, 0, unroll=False)
    for b in range(4):
        out_copy(seq - 4 + b, b).wait()


def kernel(input_ids, embed_table, proj_w, proj_b):
    bsz, seq = input_ids.shape
    d = embed_table.shape[1]
    ids_t = input_ids.T.astype(jnp.int32)
    embt = embed_table.T
    b_col = proj_b.reshape(d, 1)

    lblk = 8
    grid = seq // lblk

    log_t, = pl.pallas_call(
        _tc_body,
        grid=(grid,),
        in_specs=[
            pl.BlockSpec((lblk, bsz), lambda i: (i, 0)),
            pl.BlockSpec((d, d), lambda i: (0, 0)),
            pl.BlockSpec((d, d), lambda i: (0, 0)),
            pl.BlockSpec((d, 1), lambda i: (0, 0)),
        ],
        out_specs=[
            pl.BlockSpec((lblk, d, bsz), lambda i: (i, 0, 0)),
        ],
        out_shape=[
            jax.ShapeDtypeStruct((seq, d, bsz), _F32),
        ],
    )(ids_t, embt, proj_w, b_col)

    info = plsc.get_sparse_core_info()
    nw = info.num_cores * info.num_subcores
    b_per_w = bsz // nw
    mesh = plsc.VectorSubcoreMesh(core_axis_name="c", subcore_axis_name="s")
    lut = embt.reshape(d * d)

    sc_fn = pl.kernel(
        functools.partial(_sc_body, b_per_w, seq),
        mesh=mesh,
        out_type=jax.ShapeDtypeStruct((seq, d, bsz), _F32),
        scratch_types=[
            pltpu.VMEM((4, b_per_w), jnp.int32),
            pltpu.VMEM((4, b_per_w), jnp.int32),
            pltpu.VMEM((d * d,), _F32),
            pltpu.VMEM((d, b_per_w), _F32),
            pltpu.VMEM((d, b_per_w), _F32),
            pltpu.VMEM((d, b_per_w), _F32),
            pltpu.VMEM((d, b_per_w), _F32),
            pltpu.SemaphoreType.DMA,
            pltpu.SemaphoreType.DMA,
            pltpu.SemaphoreType.DMA,
            pltpu.SemaphoreType.DMA,
            pltpu.SemaphoreType.DMA,
            pltpu.SemaphoreType.DMA,
        ],
    )
    hid_t = sc_fn(ids_t, lut)

    hidden = jnp.transpose(hid_t, (2, 0, 1))
    logits = jnp.transpose(log_t, (2, 0, 1))
    return (logits, hidden)


# speedup vs baseline: 4.0646x; 1.0020x over previous
import functools

import jax
import jax.numpy as jnp
from jax import lax
from jax.experimental import pallas as pl
from jax.experimental.pallas import tpu as pltpu
from jax.experimental.pallas import tpu_sc as plsc

_F32 = jnp.float32


def _tc_body(ids_ref, embt_ref, pw_ref, b_ref, log_ref):
    lblk = ids_ref.shape[0]
    nb = ids_ref.shape[1]

    embt = embt_ref[...]
    ltabt = jnp.dot(pw_ref[...], embt, preferred_element_type=_F32) + b_ref[...]

    iota_v = jax.lax.broadcasted_iota(jnp.int32, (16, nb), 0)
    for l in range(lblk):
        idrow = ids_ref[pl.ds(l, 1), :]
        oh = (jnp.broadcast_to(idrow, (16, nb)) == iota_v).astype(_F32)
        log_ref[l, :, :] = jnp.dot(ltabt, oh, preferred_element_type=_F32)


def _sc_body(b_per_w, seq, ids_hbm, lut_hbm, out_hbm, ids_v0, ids_v1, lut_v,
             out_v0, out_v1, out_v2, out_v3, out_v4, out_v5, out_v6, out_v7,
             isem0, isem1, sem0, sem1, sem2, sem3, sem4, sem5, sem6, sem7):
    nc = 2
    wid = lax.axis_index("s") * nc + lax.axis_index("c")
    b0 = wid * b_per_w
    ngrp = b_per_w // 16
    nsup = seq // 8

    pltpu.sync_copy(lut_hbm, lut_v)
    lrows = [lut_v[pl.ds(16 * d, 16)] for d in range(16)]
    obufs = (out_v0, out_v1, out_v2, out_v3, out_v4, out_v5, out_v6, out_v7)
    osems = (sem0, sem1, sem2, sem3, sem4, sem5, sem6, sem7)
    ibufs = (ids_v0, ids_v1)
    isems = (isem0, isem1)

    def ids_copy(blk4, j):
        return pltpu.make_async_copy(
            ids_hbm.at[pl.ds(blk4 * 4, 4), pl.ds(b0, b_per_w)],
            ibufs[j], isems[j])

    def out_copy(l, b):
        return pltpu.make_async_copy(
            obufs[b], out_hbm.at[l, :, pl.ds(b0, b_per_w)], osems[b])

    def compute(ids_v, b, out_v):
        def g_body(g, c):
            idvec = ids_v[b, pl.ds(g * 16, 16)]
            for d in range(16):
                out_v[d, pl.ds(g * 16, 16)] = lrows[d].at[idvec].get(
                    mode="promise_in_bounds")
            return c

        lax.fori_loop(0, ngrp, g_body, 0, unroll=4)

    ids_copy(0, 0).start()

    def sup_body(i, carry):
        for j in range(2):
            blk4 = 2 * i + j
            ids_copy(blk4, j).wait()

            @pl.when(blk4 + 1 < 2 * nsup)
            def _prefetch():
                ids_copy(blk4 + 1, 1 - j).start()

            for b4 in range(4):
                buf = 4 * j + b4
                l = blk4 * 4 + b4

                @pl.when(i > 0)
                def _wait_prev():
                    out_copy(l, buf).wait()

                compute(ibufs[j], b4, obufs[buf])
                out_copy(l, buf).start()
        return carry

    lax.fori_loop(0, nsup, sup_body, 0, unroll=False)
    for b in range(8):
        out_copy(seq - 8 + b, b).wait()


def kernel(input_ids, embed_table, proj_w, proj_b):
    bsz, seq = input_ids.shape
    d = embed_table.shape[1]
    ids_t = input_ids.T.astype(jnp.int32)
    embt = embed_table.T
    b_col = proj_b.reshape(d, 1)

    lblk = 8
    grid = seq // lblk

    log_t, = pl.pallas_call(
        _tc_body,
        grid=(grid,),
        in_specs=[
            pl.BlockSpec((lblk, bsz), lambda i: (i, 0)),
            pl.BlockSpec((d, d), lambda i: (0, 0)),
            pl.BlockSpec((d, d), lambda i: (0, 0)),
            pl.BlockSpec((d, 1), lambda i: (0, 0)),
        ],
        out_specs=[
            pl.BlockSpec((lblk, d, bsz), lambda i: (i, 0, 0)),
        ],
        out_shape=[
            jax.ShapeDtypeStruct((seq, d, bsz), _F32),
        ],
    )(ids_t, embt, proj_w, b_col)

    info = plsc.get_sparse_core_info()
    nw = info.num_cores * info.num_subcores
    b_per_w = bsz // nw
    mesh = plsc.VectorSubcoreMesh(core_axis_name="c", subcore_axis_name="s")
    lut = embt.reshape(d * d)

    sc_fn = pl.kernel(
        functools.partial(_sc_body, b_per_w, seq),
        mesh=mesh,
        out_type=jax.ShapeDtypeStruct((seq, d, bsz), _F32),
        scratch_types=[
            pltpu.VMEM((4, b_per_w), jnp.int32),
            pltpu.VMEM((4, b_per_w), jnp.int32),
            pltpu.VMEM((d * d,), _F32),
            pltpu.VMEM((d, b_per_w), _F32),
            pltpu.VMEM((d, b_per_w), _F32),
            pltpu.VMEM((d, b_per_w), _F32),
            pltpu.VMEM((d, b_per_w), _F32),
            pltpu.VMEM((d, b_per_w), _F32),
            pltpu.VMEM((d, b_per_w), _F32),
            pltpu.VMEM((d, b_per_w), _F32),
            pltpu.VMEM((d, b_per_w), _F32),
            pltpu.SemaphoreType.DMA,
            pltpu.SemaphoreType.DMA,
            pltpu.SemaphoreType.DMA,
            pltpu.SemaphoreType.DMA,
            pltpu.SemaphoreType.DMA,
            pltpu.SemaphoreType.DMA,
            pltpu.SemaphoreType.DMA,
            pltpu.SemaphoreType.DMA,
            pltpu.SemaphoreType.DMA,
            pltpu.SemaphoreType.DMA,
        ],
    )
    hid_t = sc_fn(ids_t, lut)

    hidden = jnp.transpose(hid_t, (2, 0, 1))
    logits = jnp.transpose(log_t, (2, 0, 1))
    return (logits, hidden)
